# Initial kernel scaffold; baseline (speedup 1.0000x reference)
import jax
import jax.numpy as jnp
from jax.experimental import pallas as pl


def kernel(positions, query, key, cos_cache, sin_cache):
    raise NotImplementedError("write your pallas kernel here")


# speedup vs baseline: 1.5904x; 1.5904x over previous
import functools

import jax
import jax.numpy as jnp
from jax import lax
from jax.experimental import pallas as pl
from jax.experimental.pallas import tpu as pltpu
from jax.experimental.pallas import tpu_sc as plsc

HEAD_SIZE = 128
HALF = 64
NUM_Q_HEADS = 32
NUM_KV_HEADS = 8

_NC, _NS = 2, 16
_NW = _NC * _NS
_IDX_ROWS_PER_W = 2
_ROWS_PER_W = _IDX_ROWS_PER_W * 128


def _gather_body(pos_hbm, cos_hbm, sin_hbm, cos_out, sin_out,
                 idx_v, cbuf, sbuf, sem):
    wid = lax.axis_index("s") * _NC + lax.axis_index("c")
    pltpu.sync_copy(pos_hbm.at[pl.ds(wid * _IDX_ROWS_PER_W, _IDX_ROWS_PER_W)],
                    idx_v)
    copies = []
    for j in range(_IDX_ROWS_PER_W):
        copies.append(pltpu.async_copy(
            cos_hbm.at[idx_v.at[j]], cbuf.at[pl.ds(j * 128, 128)], sem))
        copies.append(pltpu.async_copy(
            sin_hbm.at[idx_v.at[j]], sbuf.at[pl.ds(j * 128, 128)], sem))
    for c in copies:
        c.wait()
    base = wid * _ROWS_PER_W
    pltpu.sync_copy(cbuf, cos_out.at[pl.ds(base, _ROWS_PER_W)])
    pltpu.sync_copy(sbuf, sin_out.at[pl.ds(base, _ROWS_PER_W)])


def _sc_gather(pos2d, cos_cache, sin_cache):
    T = pos2d.shape[0] * 128
    mesh = plsc.VectorSubcoreMesh(core_axis_name="c", subcore_axis_name="s",
                                  num_cores=_NC, num_subcores=_NS)
    f = pl.kernel(
        _gather_body,
        out_type=[jax.ShapeDtypeStruct((T, HEAD_SIZE), jnp.float32),
                  jax.ShapeDtypeStruct((T, HEAD_SIZE), jnp.float32)],
        mesh=mesh,
        scratch_types=[
            pltpu.VMEM((_IDX_ROWS_PER_W, 128), jnp.int32),
            pltpu.VMEM((_ROWS_PER_W, HEAD_SIZE), jnp.float32),
            pltpu.VMEM((_ROWS_PER_W, HEAD_SIZE), jnp.float32),
            pltpu.SemaphoreType.DMA,
        ],
    )
    return f(pos2d, cos_cache, sin_cache)


def _apply_body(cos_ref, sin_ref, q_ref, k_ref, qo_ref, ko_ref):
    c = cos_ref[...][:, None, :HALF]
    s = sin_ref[...][:, None, :HALF]
    for x_ref, o_ref in ((q_ref, qo_ref), (k_ref, ko_ref)):
        x = x_ref[...]
        x1 = x[..., :HALF]
        x2 = x[..., HALF:]
        o_ref[...] = jnp.concatenate([x1 * c - x2 * s, x2 * c + x1 * s],
                                     axis=-1)


def _tc_apply(cos_g, sin_g, q3, k3, block_t):
    T = q3.shape[0]
    grid = (T // block_t,)
    cs_spec = pl.BlockSpec((block_t, HEAD_SIZE), lambda i: (i, 0))
    q_spec = pl.BlockSpec((block_t, NUM_Q_HEADS, HEAD_SIZE),
                          lambda i: (i, 0, 0))
    k_spec = pl.BlockSpec((block_t, NUM_KV_HEADS, HEAD_SIZE),
                          lambda i: (i, 0, 0))
    return pl.pallas_call(
        _apply_body,
        grid=grid,
        in_specs=[cs_spec, cs_spec, q_spec, k_spec],
        out_specs=[q_spec, k_spec],
        out_shape=[jax.ShapeDtypeStruct(q3.shape, jnp.float32),
                   jax.ShapeDtypeStruct(k3.shape, jnp.float32)],
    )(cos_g, sin_g, q3, k3)


@jax.jit
def kernel(positions, query, key, cos_cache, sin_cache):
    T = positions.shape[0]
    pos2d = positions.astype(jnp.int32).reshape(T // 128, 128)
    cos_g, sin_g = _sc_gather(pos2d, cos_cache, sin_cache)
    q3 = query.reshape(T, NUM_Q_HEADS, HEAD_SIZE)
    k3 = key.reshape(T, NUM_KV_HEADS, HEAD_SIZE)
    qo, ko = _tc_apply(cos_g, sin_g, q3, k3, block_t=256)
    return qo.reshape(T, NUM_Q_HEADS * HEAD_SIZE), ko.reshape(T, NUM_KV_HEADS * HEAD_SIZE)


# speedup vs baseline: 4.6844x; 2.9454x over previous
import functools

import jax
import jax.numpy as jnp
from jax import lax
from jax.experimental import pallas as pl
from jax.experimental.pallas import tpu as pltpu
from jax.experimental.pallas import tpu_sc as plsc

HEAD_SIZE = 128
HALF = 64
NUM_Q_HEADS = 32
NUM_KV_HEADS = 8

_NC, _NS = 2, 16
_NW = _NC * _NS
_IDX_ROWS_PER_W = 2
_ROWS_PER_W = _IDX_ROWS_PER_W * 128


def _gather_body(pos_hbm, cos_hbm, sin_hbm, cos_out, sin_out,
                 idx_v, cbuf, sbuf, sem):
    wid = lax.axis_index("s") * _NC + lax.axis_index("c")
    pltpu.sync_copy(pos_hbm.at[pl.ds(wid * _IDX_ROWS_PER_W, _IDX_ROWS_PER_W)],
                    idx_v)
    copies = []
    for j in range(_IDX_ROWS_PER_W):
        copies.append(pltpu.async_copy(
            cos_hbm.at[idx_v.at[j]], cbuf.at[pl.ds(j * 128, 128)], sem))
        copies.append(pltpu.async_copy(
            sin_hbm.at[idx_v.at[j]], sbuf.at[pl.ds(j * 128, 128)], sem))
    for c in copies:
        c.wait()
    base = wid * _ROWS_PER_W
    pltpu.sync_copy(cbuf, cos_out.at[pl.ds(base, _ROWS_PER_W)])
    pltpu.sync_copy(sbuf, sin_out.at[pl.ds(base, _ROWS_PER_W)])


def _sc_gather(pos2d, cos_cache, sin_cache):
    T = pos2d.shape[0] * 128
    mesh = plsc.VectorSubcoreMesh(core_axis_name="c", subcore_axis_name="s",
                                  num_cores=_NC, num_subcores=_NS)
    f = pl.kernel(
        _gather_body,
        out_type=[jax.ShapeDtypeStruct((T, HEAD_SIZE), jnp.float32),
                  jax.ShapeDtypeStruct((T, HEAD_SIZE), jnp.float32)],
        mesh=mesh,
        scratch_types=[
            pltpu.VMEM((_IDX_ROWS_PER_W, 128), jnp.int32),
            pltpu.VMEM((_ROWS_PER_W, HEAD_SIZE), jnp.float32),
            pltpu.VMEM((_ROWS_PER_W, HEAD_SIZE), jnp.float32),
            pltpu.SemaphoreType.DMA,
        ],
    )
    return f(pos2d, cos_cache, sin_cache)


def _apply_body(cos_ref, sin_ref, q_ref, k_ref, qo_ref, ko_ref):
    c = cos_ref[...][:, :HALF]
    s = sin_ref[...][:, :HALF]
    cc = jnp.concatenate([c, c], axis=1)
    ss = jnp.concatenate([-s, s], axis=1)
    for x_ref, o_ref, heads in ((q_ref, qo_ref, NUM_Q_HEADS),
                                (k_ref, ko_ref, NUM_KV_HEADS)):
        for h in range(heads):
            x = x_ref[:, h * HEAD_SIZE:(h + 1) * HEAD_SIZE]
            xs = jnp.concatenate([x[:, HALF:], x[:, :HALF]], axis=1)
            o_ref[:, h * HEAD_SIZE:(h + 1) * HEAD_SIZE] = x * cc + xs * ss


def _tc_apply(cos_g, sin_g, q, k, block_t):
    T = q.shape[0]
    grid = (T // block_t,)
    cs_spec = pl.BlockSpec((block_t, HEAD_SIZE), lambda i: (i, 0))
    q_spec = pl.BlockSpec((block_t, q.shape[1]), lambda i: (i, 0))
    k_spec = pl.BlockSpec((block_t, k.shape[1]), lambda i: (i, 0))
    return pl.pallas_call(
        _apply_body,
        grid=grid,
        in_specs=[cs_spec, cs_spec, q_spec, k_spec],
        out_specs=[q_spec, k_spec],
        out_shape=[jax.ShapeDtypeStruct(q.shape, jnp.float32),
                   jax.ShapeDtypeStruct(k.shape, jnp.float32)],
        compiler_params=pltpu.CompilerParams(
            dimension_semantics=("arbitrary",)),
    )(cos_g, sin_g, q, k)


@jax.jit
def kernel(positions, query, key, cos_cache, sin_cache):
    T = positions.shape[0]
    pos2d = positions.astype(jnp.int32).reshape(T // 128, 128)
    cos_g, sin_g = _sc_gather(pos2d, cos_cache, sin_cache)
    return _tc_apply(cos_g, sin_g, query, key, block_t=256)


# speedup vs baseline: 4.7381x; 1.0114x over previous
import functools

import jax
import jax.numpy as jnp
from jax import lax
from jax.experimental import pallas as pl
from jax.experimental.pallas import tpu as pltpu
from jax.experimental.pallas import tpu_sc as plsc

HEAD_SIZE = 128
HALF = 64
NUM_Q_HEADS = 32
NUM_KV_HEADS = 8

_NC, _NS = 2, 16
_NW = _NC * _NS
_IDX_ROWS_PER_W = 2
_ROWS_PER_W = _IDX_ROWS_PER_W * 128


def _gather_body(pos_hbm, cos_hbm, sin_hbm, cos_out, sin_out,
                 idx_v, cbuf, sbuf, sem):
    wid = lax.axis_index("s") * _NC + lax.axis_index("c")
    pltpu.sync_copy(pos_hbm.at[pl.ds(wid * _IDX_ROWS_PER_W, _IDX_ROWS_PER_W)],
                    idx_v)
    copies = []
    for j in range(_IDX_ROWS_PER_W):
        copies.append(pltpu.async_copy(
            cos_hbm.at[idx_v.at[j]], cbuf.at[pl.ds(j * 128, 128)], sem))
        copies.append(pltpu.async_copy(
            sin_hbm.at[idx_v.at[j]], sbuf.at[pl.ds(j * 128, 128)], sem))
    for c in copies:
        c.wait()
    base = wid * _ROWS_PER_W
    pltpu.sync_copy(cbuf, cos_out.at[pl.ds(base, _ROWS_PER_W)])
    pltpu.sync_copy(sbuf, sin_out.at[pl.ds(base, _ROWS_PER_W)])


def _sc_gather(pos2d, cos_cache, sin_cache):
    T = pos2d.shape[0] * 128
    mesh = plsc.VectorSubcoreMesh(core_axis_name="c", subcore_axis_name="s",
                                  num_cores=_NC, num_subcores=_NS)
    f = pl.kernel(
        _gather_body,
        out_type=[jax.ShapeDtypeStruct((T, HEAD_SIZE), jnp.float32),
                  jax.ShapeDtypeStruct((T, HEAD_SIZE), jnp.float32)],
        mesh=mesh,
        scratch_types=[
            pltpu.VMEM((_IDX_ROWS_PER_W, 128), jnp.int32),
            pltpu.VMEM((_ROWS_PER_W, HEAD_SIZE), jnp.float32),
            pltpu.VMEM((_ROWS_PER_W, HEAD_SIZE), jnp.float32),
            pltpu.SemaphoreType.DMA,
        ],
    )
    return f(pos2d, cos_cache, sin_cache)


def _apply_body(cos_ref, sin_ref, q_ref, k_ref, qo_ref, ko_ref):
    c = cos_ref[...][:, :HALF]
    s = sin_ref[...][:, :HALF]
    cc = jnp.concatenate([c, c], axis=1)
    ss = jnp.concatenate([-s, s], axis=1)
    for x_ref, o_ref, heads in ((q_ref, qo_ref, NUM_Q_HEADS),
                                (k_ref, ko_ref, NUM_KV_HEADS)):
        for h in range(heads):
            x = x_ref[:, h * HEAD_SIZE:(h + 1) * HEAD_SIZE]
            xs = jnp.concatenate([x[:, HALF:], x[:, :HALF]], axis=1)
            o_ref[:, h * HEAD_SIZE:(h + 1) * HEAD_SIZE] = x * cc + xs * ss


def _tc_apply(cos_g, sin_g, q, k, block_t):
    T = q.shape[0]
    grid = (T // block_t,)
    cs_spec = pl.BlockSpec((block_t, HEAD_SIZE), lambda i: (i, 0))
    q_spec = pl.BlockSpec((block_t, q.shape[1]), lambda i: (i, 0))
    k_spec = pl.BlockSpec((block_t, k.shape[1]), lambda i: (i, 0))
    return pl.pallas_call(
        _apply_body,
        grid=grid,
        in_specs=[cs_spec, cs_spec, q_spec, k_spec],
        out_specs=[q_spec, k_spec],
        out_shape=[jax.ShapeDtypeStruct(q.shape, jnp.float32),
                   jax.ShapeDtypeStruct(k.shape, jnp.float32)],
        compiler_params=pltpu.CompilerParams(
            dimension_semantics=("arbitrary",)),
    )(cos_g, sin_g, q, k)


@jax.jit
def kernel(positions, query, key, cos_cache, sin_cache):
    T = positions.shape[0]
    pos2d = positions.astype(jnp.int32).reshape(T // 128, 128)
    cos_g, sin_g = _sc_gather(pos2d, cos_cache, sin_cache)
    return _tc_apply(cos_g, sin_g, query, key, block_t=512)
